# Initial kernel scaffold; baseline (speedup 1.0000x reference)
import functools

import jax
import jax.numpy as jnp
from jax import lax
from jax.experimental import pallas as pl
from jax.experimental.pallas import tpu as pltpu
from jax.experimental.pallas import tpu_sc as plsc

D = 64
NC = 2
NS = 16
NW = NC * NS
LANES = 16
CHUNK = 128


@functools.partial(jax.jit, static_argnums=(3,))
def _gather_add(idx, embed_table, phoneme_table, n_rows):
    b_per_w = n_rows // NW
    n_chunks = b_per_w // CHUNK
    mesh = plsc.VectorSubcoreMesh(core_axis_name="c", subcore_axis_name="s")

    @functools.partial(
        pl.kernel,
        mesh=mesh,
        out_type=jax.ShapeDtypeStruct((n_rows, D), jnp.float32),
        scratch_types=[
            pltpu.VMEM((CHUNK,), jnp.int32),
            pltpu.VMEM((CHUNK, D), jnp.float32),
            pltpu.VMEM((CHUNK, D), jnp.float32),
            pltpu.SemaphoreType.DMA,
        ],
    )
    def k(idx_hbm, embed_hbm, phon_hbm, out_hbm, idx_v, rows_e, rows_p, sem):
        wid = lax.axis_index("s") * NC + lax.axis_index("c")
        base = wid * b_per_w

        def chunk_body(g, _):
            off = base + g * CHUNK
            pltpu.sync_copy(idx_hbm.at[pl.ds(off, CHUNK)], idx_v)
            ce = pltpu.async_copy(embed_hbm.at[idx_v], rows_e, sem)
            cp = pltpu.async_copy(phon_hbm.at[idx_v], rows_p, sem)
            ce.wait()
            cp.wait()

            flat_e = rows_e.reshape(CHUNK * D)
            flat_p = rows_p.reshape(CHUNK * D)

            def add_body(i, _):
                o = i * LANES
                flat_e[pl.ds(o, LANES)] = (
                    flat_e[pl.ds(o, LANES)] + flat_p[pl.ds(o, LANES)]
                )
                return ()

            lax.fori_loop(0, CHUNK * D // LANES, add_body, ())
            pltpu.sync_copy(rows_e, out_hbm.at[pl.ds(off, CHUNK)])
            return ()

        lax.fori_loop(0, n_chunks, chunk_body, ())

    return k(idx, embed_table, phoneme_table)


def kernel(x, embed_table, phoneme_table):
    n_rows = x.shape[0] * x.shape[1]
    idx = x.reshape(n_rows).astype(jnp.int32)
    out = _gather_add(idx, embed_table, phoneme_table, n_rows)
    return out.reshape(x.shape[0], x.shape[1], D)


# speedup vs baseline: 1.9607x; 1.9607x over previous
import functools

import jax
import jax.numpy as jnp
from jax import lax
from jax.experimental import pallas as pl
from jax.experimental.pallas import tpu as pltpu
from jax.experimental.pallas import tpu_sc as plsc

D = 64
NC = 2
NS = 16
NW = NC * NS
LANES = 16
CHUNK = 128


@functools.partial(jax.jit, static_argnums=(3,))
def _gather_add(idx, embed_table, phoneme_table, n_rows):
    b_per_w = n_rows // NW
    n_chunks = b_per_w // CHUNK
    mesh = plsc.VectorSubcoreMesh(core_axis_name="c", subcore_axis_name="s")

    @functools.partial(
        pl.kernel,
        mesh=mesh,
        compiler_params=pltpu.CompilerParams(use_tc_tiling_on_sc=False),
        out_type=jax.ShapeDtypeStruct((n_rows, D), jnp.float32),
        scratch_types=[
            pltpu.VMEM((CHUNK,), jnp.int32),
            pltpu.VMEM((CHUNK, D), jnp.float32),
            pltpu.VMEM((CHUNK, D), jnp.float32),
            pltpu.SemaphoreType.DMA,
        ],
    )
    def k(idx_hbm, embed_hbm, phon_hbm, out_hbm, idx_v, rows_e, rows_p, sem):
        wid = lax.axis_index("s") * NC + lax.axis_index("c")
        base = wid * b_per_w

        def chunk_body(g, _):
            off = base + g * CHUNK
            pltpu.sync_copy(idx_hbm.at[pl.ds(off, CHUNK)], idx_v)
            ce = pltpu.async_copy(embed_hbm.at[idx_v], rows_e, sem)
            cp = pltpu.async_copy(phon_hbm.at[idx_v], rows_p, sem)
            ce.wait()
            cp.wait()

            def add_body(r, _):
                for c in range(D // LANES):
                    sl = pl.ds(c * LANES, LANES)
                    rows_e[r, sl] = rows_e[r, sl] + rows_p[r, sl]
                return ()

            lax.fori_loop(0, CHUNK, add_body, ())
            pltpu.sync_copy(rows_e, out_hbm.at[pl.ds(off, CHUNK)])
            return ()

        lax.fori_loop(0, n_chunks, chunk_body, ())

    return k(idx, embed_table, phoneme_table)


def kernel(x, embed_table, phoneme_table):
    n_rows = x.shape[0] * x.shape[1]
    idx = x.reshape(n_rows).astype(jnp.int32)
    out = _gather_add(idx, embed_table, phoneme_table, n_rows)
    return out.reshape(x.shape[0], x.shape[1], D)
